# baseline (speedup 1.0000x reference)
import functools

import jax
import jax.numpy as jnp
from jax import lax
from jax.experimental import pallas as pl
from jax.experimental.pallas import tpu as pltpu
from jax.experimental.pallas import tpu_sc as plsc

N = 10000
D = 128
E = 320000
EPS = 1e-5

NC = 2
NS = 16
NW = NC * NS
EDGES_PER_W = E // NW
CHUNK = 80
NCHUNK = EDGES_PER_W // CHUNK
PASS0 = 63
PASS1 = NCHUNK - PASS0
ZBLK = CHUNK
NB = N // ZBLK
NB_PER_TILE = -(-NB // NS)

_ROWBLK = 1000
_GRID = N // _ROWBLK


def _sc_aggregate(h, src, dst_a, dst_b):
    mesh = plsc.VectorSubcoreMesh(core_axis_name="c", subcore_axis_name="s")

    @functools.partial(
        pl.kernel,
        out_type=jax.ShapeDtypeStruct((NC, N, D), jnp.float32),
        mesh=mesh,
        scratch_types=[
            pltpu.VMEM((EDGES_PER_W,), jnp.int32),
            pltpu.VMEM((PASS0, CHUNK), jnp.int32),
            pltpu.VMEM((3, CHUNK, D), jnp.float32),
            pltpu.VMEM_SHARED((N, D), jnp.float32),
            pltpu.SemaphoreType.DMA,
            pltpu.SemaphoreType.DMA,
            pltpu.SemaphoreType.DMA,
        ],
    )
    def k(h_hbm, src_hbm, dsta_hbm, dstb_hbm, out_hbm,
          ids, idd, rowsb, acc, sg0, sg1, sg2):
        rows = [rowsb.at[0], rowsb.at[1], rowsb.at[2]]
        sg = [sg0, sg1, sg2]
        c = lax.axis_index("c")
        s = lax.axis_index("s")
        wid = c * NS + s

        @pl.loop(0, ZBLK)
        def _(r):
            @pl.loop(0, D // 16)
            def _(j):
                rowsb[0, r, pl.ds(j * 16, 16)] = jnp.zeros((16,), jnp.float32)

        @pl.loop(0, NB_PER_TILE)
        def _(j):
            blk = s + j * NS

            @pl.when(blk < NB)
            def _():
                pltpu.sync_copy(rows[0], acc.at[pl.ds(blk * ZBLK, ZBLK)])

        pltpu.sync_copy(src_hbm.at[wid], ids)

        plsc.subcore_barrier()

        def gidx(base, j):
            return ids.at[pl.ds((base + j) * CHUNK, CHUNK)]

        def edge_pass(base, n_p):
            for b in range(3):
                pltpu.async_copy(h_hbm.at[gidx(base, b)], rows[b], sg[b])

            @pl.loop(0, n_p, step=3)
            def _(i):
                for b in range(3):
                    @pl.when(i + b < n_p)
                    def _(b=b):
                        pltpu.make_async_copy(
                            h_hbm.at[gidx(base, i + b)], rows[b],
                            sg[b]).wait()
                        pltpu.sync_copy(rows[b], acc.at[idd.at[i + b]],
                                        add=True)

                        @pl.when(i + b + 3 < n_p)
                        def _(b=b):
                            pltpu.async_copy(
                                h_hbm.at[gidx(base, i + b + 3)], rows[b],
                                sg[b])

        pltpu.sync_copy(dsta_hbm.at[wid], idd)
        edge_pass(0, PASS0)
        pltpu.sync_copy(dstb_hbm.at[wid], idd.at[pl.ds(0, PASS1)])
        edge_pass(PASS0, PASS1)

        plsc.subcore_barrier()

        @pl.loop(0, NB_PER_TILE)
        def _(j):
            blk = s + j * NS

            @pl.when(blk < NB)
            def _():
                r0 = blk * ZBLK
                pltpu.sync_copy(acc.at[pl.ds(r0, ZBLK)], rows[0])
                pltpu.sync_copy(rows[0], out_hbm.at[c, pl.ds(r0, ZBLK)])

    return k(h, src, dst_a, dst_b)


def _tc_mm(x, W):
    def body(x_ref, w_ref, o_ref):
        o_ref[...] = jnp.dot(x_ref[...], w_ref[...],
                             preferred_element_type=jnp.float32)

    return pl.pallas_call(
        body,
        out_shape=jax.ShapeDtypeStruct((N, D), jnp.float32),
        grid=(_GRID,),
        in_specs=[pl.BlockSpec((_ROWBLK, D), lambda i: (i, 0)),
                  pl.BlockSpec((D, D), lambda i: (0, 0))],
        out_specs=pl.BlockSpec((_ROWBLK, D), lambda i: (i, 0)),
    )(x, W)


def _tc_layer1(p, b0, gamma, beta, run_mean, run_var, W1):
    def body(p_ref, b_ref, g_ref, be_ref, m_ref, v_ref, w_ref, o_ref):
        y = p_ref[0] + p_ref[1] + b_ref[...]
        scale = g_ref[...] * lax.rsqrt(v_ref[...] + EPS)
        y = (y - m_ref[...]) * scale + be_ref[...]
        y = jnp.maximum(y, 0.0)
        o_ref[...] = jnp.dot(y, w_ref[...], preferred_element_type=jnp.float32)

    vec = pl.BlockSpec((1, D), lambda i: (0, 0))
    return pl.pallas_call(
        body,
        out_shape=jax.ShapeDtypeStruct((N, D), jnp.float32),
        grid=(_GRID,),
        in_specs=[pl.BlockSpec((NC, _ROWBLK, D), lambda i: (0, i, 0)),
                  vec, vec, vec, vec, vec,
                  pl.BlockSpec((D, D), lambda i: (0, 0))],
        out_specs=pl.BlockSpec((_ROWBLK, D), lambda i: (i, 0)),
    )(p, b0.reshape(1, D), gamma.reshape(1, D), beta.reshape(1, D),
      run_mean.reshape(1, D), run_var.reshape(1, D), W1)


def _tc_combine(p, b1):
    def body(p_ref, b_ref, o_ref):
        o_ref[...] = p_ref[0] + p_ref[1] + b_ref[...]

    return pl.pallas_call(
        body,
        out_shape=jax.ShapeDtypeStruct((N, D), jnp.float32),
        grid=(_GRID,),
        in_specs=[pl.BlockSpec((NC, _ROWBLK, D), lambda i: (0, i, 0)),
                  pl.BlockSpec((1, D), lambda i: (0, 0))],
        out_specs=pl.BlockSpec((_ROWBLK, D), lambda i: (i, 0)),
    )(p, b1.reshape(1, D))


def kernel(x, edge_index, W0, b0, W1, b1, gamma, beta, run_mean, run_var):
    cut = PASS0 * CHUNK
    src = edge_index[0].reshape(NW, EDGES_PER_W)
    dst = edge_index[1].reshape(NW, EDGES_PER_W)
    dst_a = dst[:, :cut].reshape(NW, PASS0, CHUNK)
    dst_b = dst[:, cut:].reshape(NW, PASS1, CHUNK)
    h0 = _tc_mm(x, W0)
    p0 = _sc_aggregate(h0, src, dst_a, dst_b)
    h1 = _tc_layer1(p0, b0, gamma, beta, run_mean, run_var, W1)
    p1 = _sc_aggregate(h1, src, dst_a, dst_b)
    return _tc_combine(p1, b1)


# speedup vs baseline: 1.0297x; 1.0297x over previous
import functools

import jax
import jax.numpy as jnp
from jax import lax
from jax.experimental import pallas as pl
from jax.experimental.pallas import tpu as pltpu
from jax.experimental.pallas import tpu_sc as plsc

N = 10000
D = 128
E = 320000
EPS = 1e-5

NC = 2
NS = 16
NW = NC * NS
EDGES_PER_W = E // NW
CHUNK = 80
NCHUNK = EDGES_PER_W // CHUNK
PASS0 = 63
PASS1 = NCHUNK - PASS0
ZBLK = CHUNK
NB = N // ZBLK
NB_PER_TILE = -(-NB // NS)

_ROWBLK = 1000
_GRID = N // _ROWBLK


def _sc_aggregate(h, src, dst_a, dst_b):
    mesh = plsc.VectorSubcoreMesh(core_axis_name="c", subcore_axis_name="s")

    @functools.partial(
        pl.kernel,
        out_type=jax.ShapeDtypeStruct((NC, N, D), jnp.float32),
        mesh=mesh,
        scratch_types=[
            pltpu.VMEM((EDGES_PER_W,), jnp.int32),
            pltpu.VMEM((PASS0, CHUNK), jnp.int32),
            pltpu.VMEM((3, CHUNK, D), jnp.float32),
            pltpu.VMEM_SHARED((N, D), jnp.float32),
            pltpu.SemaphoreType.DMA,
            pltpu.SemaphoreType.DMA,
            pltpu.SemaphoreType.DMA,
            pltpu.SemaphoreType.DMA,
        ],
    )
    def k(h_hbm, src_hbm, dsta_hbm, dstb_hbm, out_hbm,
          ids, idd, rowsb, acc, sg0, sg1, sg2, si):
        rows = [rowsb.at[0], rowsb.at[1], rowsb.at[2]]
        sg = [sg0, sg1, sg2]
        c = lax.axis_index("c")
        s = lax.axis_index("s")
        wid = c * NS + s

        pltpu.async_copy(src_hbm.at[wid], ids, si)
        pltpu.async_copy(dsta_hbm.at[wid], idd, sg0)

        @pl.loop(0, ZBLK)
        def _(r):
            @pl.loop(0, D // 16)
            def _(j):
                rowsb[0, r, pl.ds(j * 16, 16)] = jnp.zeros((16,), jnp.float32)

        @pl.loop(0, NB_PER_TILE)
        def _(j):
            blk = s + j * NS

            @pl.when(blk < NB)
            def _():
                pltpu.sync_copy(rows[0], acc.at[pl.ds(blk * ZBLK, ZBLK)])

        pltpu.make_async_copy(src_hbm.at[wid], ids, si).wait()
        pltpu.make_async_copy(dsta_hbm.at[wid], idd, sg0).wait()

        plsc.subcore_barrier()

        def gidx(base, j):
            return ids.at[pl.ds((base + j) * CHUNK, CHUNK)]

        def edge_pass(base, n_p):
            for b in range(3):
                pltpu.async_copy(h_hbm.at[gidx(base, b)], rows[b], sg[b])

            @pl.loop(0, n_p, step=3)
            def _(i):
                for b in range(3):
                    @pl.when(i + b < n_p)
                    def _(b=b):
                        pltpu.make_async_copy(
                            h_hbm.at[gidx(base, i + b)], rows[b],
                            sg[b]).wait()
                        pltpu.sync_copy(rows[b], acc.at[idd.at[i + b]],
                                        add=True)

                        @pl.when(i + b + 3 < n_p)
                        def _(b=b):
                            pltpu.async_copy(
                                h_hbm.at[gidx(base, i + b + 3)], rows[b],
                                sg[b])

        edge_pass(0, PASS0)
        pltpu.sync_copy(dstb_hbm.at[wid], idd.at[pl.ds(0, PASS1)])
        edge_pass(PASS0, PASS1)

        plsc.subcore_barrier()

        @pl.loop(0, NB_PER_TILE)
        def _(j):
            blk = s + j * NS

            @pl.when(blk < NB)
            def _():
                r0 = blk * ZBLK
                pltpu.sync_copy(acc.at[pl.ds(r0, ZBLK)],
                                out_hbm.at[c, pl.ds(r0, ZBLK)])

    return k(h, src, dst_a, dst_b)


def _tc_mm(x, W):
    def body(x_ref, w_ref, o_ref):
        o_ref[...] = jnp.dot(x_ref[...], w_ref[...],
                             preferred_element_type=jnp.float32)

    return pl.pallas_call(
        body,
        out_shape=jax.ShapeDtypeStruct((N, D), jnp.float32),
        grid=(_GRID,),
        in_specs=[pl.BlockSpec((_ROWBLK, D), lambda i: (i, 0)),
                  pl.BlockSpec((D, D), lambda i: (0, 0))],
        out_specs=pl.BlockSpec((_ROWBLK, D), lambda i: (i, 0)),
    )(x, W)


def _tc_layer1(p, b0, gamma, beta, run_mean, run_var, W1):
    def body(p_ref, b_ref, g_ref, be_ref, m_ref, v_ref, w_ref, o_ref):
        y = p_ref[0] + p_ref[1] + b_ref[...]
        scale = g_ref[...] * lax.rsqrt(v_ref[...] + EPS)
        y = (y - m_ref[...]) * scale + be_ref[...]
        y = jnp.maximum(y, 0.0)
        o_ref[...] = jnp.dot(y, w_ref[...], preferred_element_type=jnp.float32)

    vec = pl.BlockSpec((1, D), lambda i: (0, 0))
    return pl.pallas_call(
        body,
        out_shape=jax.ShapeDtypeStruct((N, D), jnp.float32),
        grid=(_GRID,),
        in_specs=[pl.BlockSpec((NC, _ROWBLK, D), lambda i: (0, i, 0)),
                  vec, vec, vec, vec, vec,
                  pl.BlockSpec((D, D), lambda i: (0, 0))],
        out_specs=pl.BlockSpec((_ROWBLK, D), lambda i: (i, 0)),
    )(p, b0.reshape(1, D), gamma.reshape(1, D), beta.reshape(1, D),
      run_mean.reshape(1, D), run_var.reshape(1, D), W1)


def _tc_combine(p, b1):
    def body(p_ref, b_ref, o_ref):
        o_ref[...] = p_ref[0] + p_ref[1] + b_ref[...]

    return pl.pallas_call(
        body,
        out_shape=jax.ShapeDtypeStruct((N, D), jnp.float32),
        grid=(_GRID,),
        in_specs=[pl.BlockSpec((NC, _ROWBLK, D), lambda i: (0, i, 0)),
                  pl.BlockSpec((1, D), lambda i: (0, 0))],
        out_specs=pl.BlockSpec((_ROWBLK, D), lambda i: (i, 0)),
    )(p, b1.reshape(1, D))


def kernel(x, edge_index, W0, b0, W1, b1, gamma, beta, run_mean, run_var):
    cut = PASS0 * CHUNK
    src = edge_index[0].reshape(NW, EDGES_PER_W)
    dst = edge_index[1].reshape(NW, EDGES_PER_W)
    dst_a = dst[:, :cut].reshape(NW, PASS0, CHUNK)
    dst_b = dst[:, cut:].reshape(NW, PASS1, CHUNK)
    h0 = _tc_mm(x, W0)
    p0 = _sc_aggregate(h0, src, dst_a, dst_b)
    h1 = _tc_layer1(p0, b0, gamma, beta, run_mean, run_var, W1)
    p1 = _sc_aggregate(h1, src, dst_a, dst_b)
    return _tc_combine(p1, b1)
